# Initial kernel scaffold; baseline (speedup 1.0000x reference)
import jax
import jax.numpy as jnp
from jax.experimental import pallas as pl


def kernel(action, object, location, edge_index, batch, A_tab, O_tab, L_tab, W_in, b_in, W1, b1, g1, be1, W2, b2, g2, be2, W3, b3, g3, be3, Wc1, bc1, Wc2, bc2, Wc3, bc3):
    raise NotImplementedError("write your pallas kernel here")


# speedup vs baseline: 4.6396x; 4.6396x over previous
import functools

import jax
import jax.numpy as jnp
from jax import lax
from jax.experimental import pallas as pl
from jax.experimental.pallas import tpu as pltpu
from jax.experimental.pallas import tpu_sc as plsc

N = 10000
NPAD = 10240
E = 320000
NA = 200
NO = 5000
NL = 1000
NC = 50
ED = 32
HD = 128
B = 64
NT = NA + NO + NL

NCORE = 2
NSUB = 16
NWORK = NCORE * NSUB
CH = 128
EPW = NPAD
EPAD = NWORK * EPW
CPW = EPW // CH
NCHE = EPAD // CH
FPAD = 32768
FCH = FPAD // CH
FCPW = FCH // NWORK
RPS = NPAD // NSUB

_f32 = jnp.float32
_i32 = jnp.int32

_sc_mesh = functools.partial(
    plsc.VectorSubcoreMesh, core_axis_name="c", subcore_axis_name="s",
    num_cores=NCORE, num_subcores=NSUB)


def _tables_body(a_ref, o_ref, l_ref, win_ref, bin_ref, t_ref):
    wa = win_ref[0:ED, :]
    wo = win_ref[ED:2 * ED, :]
    wl = win_ref[2 * ED:3 * ED, :]
    woh = win_ref[3 * ED:3 * ED + NA, :]
    t_ref[0:NA, :] = (jnp.dot(a_ref[...], wa, preferred_element_type=_f32)
                      + woh + bin_ref[...])
    t_ref[NA:NA + NO, :] = jnp.dot(o_ref[...], wo, preferred_element_type=_f32)
    t_ref[NA + NO:NT, :] = jnp.dot(l_ref[...], wl, preferred_element_type=_f32)


def _tc_tables(a_tab, o_tab, l_tab, w_in, b_in):
    return pl.pallas_call(
        _tables_body,
        out_shape=jax.ShapeDtypeStruct((NT, HD), _f32),
    )(a_tab, o_tab, l_tab, w_in, b_in)


def _sc1_body(t_hbm, idxf_hbm, feat_hbm, gidx, rbg, g0, g1):
    c = lax.axis_index("c")
    s = lax.axis_index("s")
    w = c * NSUB + s

    pltpu.sync_copy(idxf_hbm.at[pl.ds(w * FCPW, FCPW)], gidx)
    pltpu.async_copy(t_hbm.at[gidx.at[0, 0]], rbg.at[0], g0)
    for k in range(FCPW):
        p = k % 2
        sem = g0 if p == 0 else g1
        osem = g1 if p == 0 else g0
        pltpu.make_async_copy(t_hbm.at[gidx.at[k, 0]], rbg.at[p], sem).wait()
        if k < FCPW - 1:
            pltpu.async_copy(t_hbm.at[gidx.at[k + 1, 0]], rbg.at[1 - p], osem)
        pltpu.sync_copy(rbg.at[p], feat_hbm.at[pl.ds(w * FCPW * CH + k * CH, CH)])


def _sc_feat(t, idxf):
    kfn = pl.kernel(
        _sc1_body,
        out_type=jax.ShapeDtypeStruct((FPAD, HD), _f32),
        mesh=_sc_mesh(),
        scratch_types=[
            pltpu.VMEM((FCPW, 1, CH), _i32),
            pltpu.VMEM((2, CH, HD), _f32),
            pltpu.SemaphoreType.DMA,
            pltpu.SemaphoreType.DMA,
        ],
    )
    return kfn(t, idxf)


def _x0_body(xa_ref, xo_ref, xl_ref, d0_ref, d1_ref, w1_ref,
             x0_ref, g1_ref, dinv_ref):
    deg = d0_ref[0, :, 0:1] + d1_ref[0, :, 0:1] + 1.0
    dv = 1.0 / jnp.sqrt(deg)
    x0 = jnp.maximum(xa_ref[...] + xo_ref[...] + xl_ref[...], 0.0)
    x0_ref[...] = x0
    dinv_ref[...] = dv
    g1_ref[...] = jnp.dot(x0, w1_ref[...], preferred_element_type=_f32) * dv


def _tc_x0(feat, deg2, w1):
    blk = 1024
    nblk = NPAD // blk
    return pl.pallas_call(
        _x0_body,
        grid=(nblk,),
        in_specs=[
            pl.BlockSpec((blk, HD), lambda i: (i, 0)),
            pl.BlockSpec((blk, HD), lambda i: (i + nblk, 0)),
            pl.BlockSpec((blk, HD), lambda i: (i + 2 * nblk, 0)),
            pl.BlockSpec((1, blk, HD), lambda i: (0, i, 0)),
            pl.BlockSpec((1, blk, HD), lambda i: (1, i, 0)),
            pl.BlockSpec((HD, HD), lambda i: (0, 0)),
        ],
        out_specs=[
            pl.BlockSpec((blk, HD), lambda i: (i, 0)),
            pl.BlockSpec((blk, HD), lambda i: (i, 0)),
            pl.BlockSpec((blk, 1), lambda i: (i, 0)),
        ],
        out_shape=[
            jax.ShapeDtypeStruct((NPAD, HD), _f32),
            jax.ShapeDtypeStruct((NPAD, HD), _f32),
            jax.ShapeDtypeStruct((NPAD, 1), _f32),
        ],
    )(feat, feat, feat, deg2, deg2, w1)


def _agg_body(g_hbm, esrc_hbm, edst_hbm, acc_hbm,
              zb, ibs, ibd, rb, accsh, g0, g1, s0, s1):
    c = lax.axis_index("c")
    s = lax.axis_index("s")

    @pl.loop(0, 16)
    def _(i):
        @pl.loop(0, HD // 16)
        def _(k):
            zb[i, pl.ds(k * 16, 16)] = jnp.zeros((16,), _f32)

    @pl.loop(0, RPS // 16)
    def _(r):
        pltpu.sync_copy(zb, accsh.at[pl.ds(s * RPS + r * 16, 16)])

    plsc.subcore_barrier()

    ebase0 = (c * (NCHE // 2) + s * CPW) * CH

    @pl.loop(0, CPW)
    def _(j):
        pltpu.sync_copy(esrc_hbm.at[pl.ds(ebase0 + j * CH, CH)], ibs)
        pltpu.sync_copy(edst_hbm.at[pl.ds(ebase0 + j * CH, CH)], ibd)
        pltpu.sync_copy(g_hbm.at[ibs], rb)
        pltpu.sync_copy(rb, accsh.at[ibd], add=True)

    plsc.subcore_barrier()

    @pl.loop(0, RPS // CH)
    def _(r):
        pltpu.sync_copy(accsh.at[pl.ds(s * RPS + r * CH, CH)],
                        acc_hbm.at[pl.ds(c * NPAD + s * RPS + r * CH, CH)])


def _sc_agg(g, esrc, edst):
    kfn = pl.kernel(
        _agg_body,
        out_type=jax.ShapeDtypeStruct((NCORE * NPAD, HD), _f32),
        mesh=_sc_mesh(),
        scratch_types=[
            pltpu.VMEM((16, HD), _f32),
            pltpu.VMEM((CH,), _i32),
            pltpu.VMEM((CH,), _i32),
            pltpu.VMEM((CH, HD), _f32),
            pltpu.VMEM_SHARED((NPAD, HD), _f32),
            pltpu.SemaphoreType.DMA,
            pltpu.SemaphoreType.DMA,
            pltpu.SemaphoreType.DMA,
            pltpu.SemaphoreType.DMA,
        ],
    )
    return kfn(g, esrc, edst).reshape(NCORE, NPAD, HD)


def _post_body(a0_ref, a1_ref, g_ref, xp_ref, dv_ref, b_ref, gam_ref, bet_ref,
               wn_ref, xn_ref, gn_ref):
    dv = dv_ref[...]
    agg = a0_ref[0] + a1_ref[0] + g_ref[...]
    pre = agg * dv + b_ref[...]
    m = jnp.mean(pre, axis=-1, keepdims=True)
    cen = pre - m
    v = jnp.mean(cen * cen, axis=-1, keepdims=True)
    ln = cen / jnp.sqrt(v + 1e-5) * gam_ref[...] + bet_ref[...]
    xn = xp_ref[...] + jnp.maximum(ln, 0.0)
    xn_ref[...] = xn
    if wn_ref is not None:
        gn_ref[...] = jnp.dot(xn, wn_ref[...], preferred_element_type=_f32) * dv


def _post_body_last(a0_ref, a1_ref, g_ref, xp_ref, dv_ref, b_ref, gam_ref,
                    bet_ref, xn_ref):
    _post_body(a0_ref, a1_ref, g_ref, xp_ref, dv_ref, b_ref, gam_ref, bet_ref,
               None, xn_ref, None)


def _tc_post(acc, g, xp, dinv, b, gam, bet, wn=None):
    blk = 1024
    nblk = NPAD // blk
    in_specs = [
        pl.BlockSpec((1, blk, HD), lambda i: (0, i, 0)),
        pl.BlockSpec((1, blk, HD), lambda i: (1, i, 0)),
        pl.BlockSpec((blk, HD), lambda i: (i, 0)),
        pl.BlockSpec((blk, HD), lambda i: (i, 0)),
        pl.BlockSpec((blk, 1), lambda i: (i, 0)),
        pl.BlockSpec((1, HD), lambda i: (0, 0)),
        pl.BlockSpec((1, HD), lambda i: (0, 0)),
        pl.BlockSpec((1, HD), lambda i: (0, 0)),
    ]
    args = [acc, acc, g, xp, dinv, b, gam, bet]
    if wn is None:
        return pl.pallas_call(
            _post_body_last,
            grid=(nblk,),
            in_specs=in_specs,
            out_specs=pl.BlockSpec((blk, HD), lambda i: (i, 0)),
            out_shape=jax.ShapeDtypeStruct((NPAD, HD), _f32),
        )(*args)
    in_specs.append(pl.BlockSpec((HD, HD), lambda i: (0, 0)))
    args.append(wn)
    return pl.pallas_call(
        _post_body,
        grid=(nblk,),
        in_specs=in_specs,
        out_specs=[
            pl.BlockSpec((blk, HD), lambda i: (i, 0)),
            pl.BlockSpec((blk, HD), lambda i: (i, 0)),
        ],
        out_shape=[
            jax.ShapeDtypeStruct((NPAD, HD), _f32),
            jax.ShapeDtypeStruct((NPAD, HD), _f32),
        ],
    )(*args)


_PBLK = 1000
_NPB = N // _PBLK


def _pool_body(x_ref, bat_ref, act_ref, wc1_ref, bc1_ref, wc2_ref, bc2_ref,
               wc3_ref, bc3_ref, logits_ref, xg_ref,
               xsum_s, xmax_s, ag_s, cnt_s):
    i = pl.program_id(0)

    @pl.when(i == 0)
    def _():
        xsum_s[...] = jnp.zeros((B, HD), _f32)
        xmax_s[...] = jnp.full((B, HD), -jnp.inf, _f32)
        ag_s[...] = jnp.zeros((B, NA), _f32)
        cnt_s[...] = jnp.zeros((B, 1), _f32)

    x = x_ref[...]
    bat = bat_ref[...]
    sf = (bat == lax.broadcasted_iota(_i32, (_PBLK, B), 1)).astype(_f32)
    ohf = (act_ref[...] == lax.broadcasted_iota(_i32, (_PBLK, NA), 1)).astype(_f32)
    dn = (((0,), (0,)), ((), ()))
    xsum_s[...] += lax.dot_general(sf, x, dn, preferred_element_type=_f32)
    ag_s[...] += lax.dot_general(sf, ohf, dn, preferred_element_type=_f32)
    cnt_s[...] += lax.dot_general(sf, jnp.ones((_PBLK, 1), _f32), dn,
                                  preferred_element_type=_f32)

    b0 = bat_ref[0, 0]
    b1 = bat_ref[_PBLK - 1, 0]

    def seg_body(b, _):
        vals = jnp.where(bat == b, x, -jnp.inf)
        mx = jnp.max(vals, axis=0, keepdims=True)
        xmax_s[pl.ds(b, 1), :] = jnp.maximum(xmax_s[pl.ds(b, 1), :], mx)
        return 0

    lax.fori_loop(b0, b1 + 1, seg_body, 0)

    @pl.when(i == _NPB - 1)
    def _():
        cnt = cnt_s[...]
        nonempty = cnt > 0.0
        xmean = xsum_s[...] / jnp.maximum(cnt, 1.0)
        xmax = xmax_s[...]
        agf = jnp.where(nonempty, (ag_s[...] > 0.0).astype(_f32), -jnp.inf)
        xg_ref[:, 0:HD] = xmean
        xg_ref[:, HD:2 * HD] = xmax
        xg_ref[:, 2 * HD:2 * HD + NA] = agf
        h1 = jnp.maximum(
            lax.dot_general(xmean, wc1_ref[0:HD, :], (((1,), (0,)), ((), ())),
                            preferred_element_type=_f32)
            + lax.dot_general(xmax, wc1_ref[HD:2 * HD, :],
                              (((1,), (0,)), ((), ())),
                              preferred_element_type=_f32)
            + lax.dot_general(agf, wc1_ref[2 * HD:2 * HD + NA, :],
                              (((1,), (0,)), ((), ())),
                              preferred_element_type=_f32)
            + bc1_ref[...], 0.0)
        h2 = jnp.maximum(
            jnp.dot(h1, wc2_ref[...], preferred_element_type=_f32)
            + bc2_ref[...], 0.0)
        logits_ref[...] = (jnp.dot(h2, wc3_ref[...], preferred_element_type=_f32)
                           + bc3_ref[...])


def _tc_pool_head(x3, bat2, act2, wc1, bc1, wc2, bc2, wc3, bc3):
    return pl.pallas_call(
        _pool_body,
        grid=(_NPB,),
        in_specs=[
            pl.BlockSpec((_PBLK, HD), lambda i: (i, 0)),
            pl.BlockSpec((_PBLK, 1), lambda i: (i, 0)),
            pl.BlockSpec((_PBLK, 1), lambda i: (i, 0)),
            pl.BlockSpec((2 * HD + NA, HD), lambda i: (0, 0)),
            pl.BlockSpec((1, HD), lambda i: (0, 0)),
            pl.BlockSpec((HD, HD // 2), lambda i: (0, 0)),
            pl.BlockSpec((1, HD // 2), lambda i: (0, 0)),
            pl.BlockSpec((HD // 2, NC), lambda i: (0, 0)),
            pl.BlockSpec((1, NC), lambda i: (0, 0)),
        ],
        out_specs=[
            pl.BlockSpec((B, NC), lambda i: (0, 0)),
            pl.BlockSpec((B, 2 * HD + NA), lambda i: (0, 0)),
        ],
        out_shape=[
            jax.ShapeDtypeStruct((B, NC), _f32),
            jax.ShapeDtypeStruct((B, 2 * HD + NA), _f32),
        ],
        scratch_shapes=[
            pltpu.VMEM((B, HD), _f32),
            pltpu.VMEM((B, HD), _f32),
            pltpu.VMEM((B, NA), _f32),
            pltpu.VMEM((B, 1), _f32),
        ],
    )(x3, bat2, act2, wc1, bc1, wc2, bc2, wc3, bc3)


def kernel(action, object, location, edge_index, batch,
           A_tab, O_tab, L_tab, W_in, b_in,
           W1, b1, g1, be1, W2, b2, g2, be2, W3, b3, g3, be3,
           Wc1, bc1, Wc2, bc2, Wc3, bc3):
    action = action.astype(_i32)
    object = object.astype(_i32)
    location = location.astype(_i32)
    batch = batch.astype(_i32)
    edge_index = edge_index.astype(_i32)

    pad_e = EPAD - E
    esrc = jnp.concatenate([edge_index[0], jnp.full((pad_e,), N, _i32)])
    edst = jnp.concatenate([edge_index[1], jnp.full((pad_e,), NPAD - 1, _i32)])

    padn = NPAD - N
    ia = jnp.concatenate([action, jnp.zeros((padn,), _i32)])
    io = jnp.concatenate([object + NA, jnp.zeros((padn,), _i32)])
    il = jnp.concatenate([location + NA + NO, jnp.zeros((padn,), _i32)])
    idxf = jnp.concatenate([ia, io, il, jnp.zeros((FPAD - 3 * NPAD,), _i32)])
    idxf = idxf.reshape(FCH, 1, CH)

    bat2 = batch.reshape(N, 1)
    act2 = action.reshape(N, 1)
    b_in2 = b_in.reshape(1, HD)
    b1_2, gam1, bet1 = b1.reshape(1, HD), g1.reshape(1, HD), be1.reshape(1, HD)
    b2_2, gam2, bet2 = b2.reshape(1, HD), g2.reshape(1, HD), be2.reshape(1, HD)
    b3_2, gam3, bet3 = b3.reshape(1, HD), g3.reshape(1, HD), be3.reshape(1, HD)
    bc1_2 = bc1.reshape(1, HD)
    bc2_2 = bc2.reshape(1, HD // 2)
    bc3_2 = bc3.reshape(1, NC)

    t = _tc_tables(A_tab, O_tab, L_tab, W_in, b_in2)
    feat = _sc_feat(t, idxf)
    deg2 = _sc_agg(jnp.ones((NPAD, HD), _f32), esrc, edst)
    x0, gm1, dinv = _tc_x0(feat, deg2, W1)
    acc1 = _sc_agg(gm1, esrc, edst)
    x1, gm2 = _tc_post(acc1, gm1, x0, dinv, b1_2, gam1, bet1, W2)
    acc2 = _sc_agg(gm2, esrc, edst)
    x2, gm3 = _tc_post(acc2, gm2, x1, dinv, b2_2, gam2, bet2, W3)
    acc3 = _sc_agg(gm3, esrc, edst)
    x3 = _tc_post(acc3, gm3, x2, dinv, b3_2, gam3, bet3)
    logits, x_graph = _tc_pool_head(x3, bat2, act2, Wc1, bc1_2, Wc2, bc2_2,
                                    Wc3, bc3_2)
    return (logits, x_graph)
